# Initial kernel scaffold; baseline (speedup 1.0000x reference)
import jax
import jax.numpy as jnp
from jax.experimental import pallas as pl


def kernel(x, user_emb, movie_emb, W1, b1, W2, b2):
    raise NotImplementedError("write your pallas kernel here")


# speedup vs baseline: 1.0966x; 1.0966x over previous
import functools

import jax
import jax.numpy as jnp
from jax import lax
from jax.experimental import pallas as pl
from jax.experimental.pallas import tpu as pltpu
from jax.experimental.pallas import tpu_sc as plsc

_B = 16384
_D = 64
_NW = 32
_BPW = _B // _NW
_CHUNK = 128
_NCH = _BPW // _CHUNK

_BBLK = 2048
_H1 = 256


def _sc_gather(users, movies, user_emb, movie_emb):
    mesh = plsc.VectorSubcoreMesh(core_axis_name="c", subcore_axis_name="s")

    @functools.partial(
        pl.kernel,
        mesh=mesh,
        out_type=[
            jax.ShapeDtypeStruct((_B, _D), jnp.float32),
            jax.ShapeDtypeStruct((_B, _D), jnp.float32),
        ],
        scratch_types=[
            pltpu.VMEM((_NCH, _CHUNK), jnp.int32),
            pltpu.VMEM((_NCH, _CHUNK), jnp.int32),
            pltpu.VMEM((_BPW, _D), jnp.float32),
            pltpu.VMEM((_BPW, _D), jnp.float32),
            pltpu.SemaphoreType.DMA,
            pltpu.SemaphoreType.DMA,
        ],
        compiler_params=pltpu.CompilerParams(use_tc_tiling_on_sc=False),
    )
    def k(users_hbm, movies_hbm, uemb_hbm, memb_hbm, out_u, out_m,
          uidx_v, midx_v, urows_v, mrows_v, sem_u, sem_m):
        wid = lax.axis_index("s") * 2 + lax.axis_index("c")
        base = wid * _BPW
        pltpu.sync_copy(users_hbm.at[wid], uidx_v)
        pltpu.sync_copy(movies_hbm.at[wid], midx_v)
        handles = []
        for j in range(_NCH):
            handles.append(pltpu.async_copy(
                uemb_hbm.at[uidx_v.at[j]],
                urows_v.at[pl.ds(j * _CHUNK, _CHUNK)], sem_u))
            handles.append(pltpu.async_copy(
                memb_hbm.at[midx_v.at[j]],
                mrows_v.at[pl.ds(j * _CHUNK, _CHUNK)], sem_m))
        for h in handles:
            h.wait()
        pltpu.sync_copy(urows_v, out_u.at[pl.ds(base, _BPW)])
        pltpu.sync_copy(mrows_v, out_m.at[pl.ds(base, _BPW)])

    return k(users, movies, user_emb, movie_emb)


def _mlp_body(u_ref, m_ref, w1u_ref, w1m_ref, b1_ref, w2_ref, b2_ref, out_ref):
    h = (jnp.dot(u_ref[...], w1u_ref[...], preferred_element_type=jnp.float32)
         + jnp.dot(m_ref[...], w1m_ref[...], preferred_element_type=jnp.float32)
         + b1_ref[...])
    h = jnp.maximum(h, 0.0)
    res = lax.dot_general(h, w2_ref[...], (((1,), (1,)), ((), ())),
                          preferred_element_type=jnp.float32)
    out_ref[...] = res[:, 0] + b2_ref[0, 0]


def _tc_mlp(u_rows, m_rows, w1u_t, w1m_t, b1, w2, b2):
    grid = (_B // _BBLK,)
    return pl.pallas_call(
        _mlp_body,
        grid=grid,
        in_specs=[
            pl.BlockSpec((_BBLK, _D), lambda i: (i, 0)),
            pl.BlockSpec((_BBLK, _D), lambda i: (i, 0)),
            pl.BlockSpec((_D, _H1), lambda i: (0, 0)),
            pl.BlockSpec((_D, _H1), lambda i: (0, 0)),
            pl.BlockSpec((1, _H1), lambda i: (0, 0)),
            pl.BlockSpec((1, _H1), lambda i: (0, 0)),
            pl.BlockSpec((1, 1), lambda i: (0, 0)),
        ],
        out_specs=pl.BlockSpec((_BBLK,), lambda i: (i,)),
        out_shape=jax.ShapeDtypeStruct((_B,), jnp.float32),
        compiler_params=pltpu.CompilerParams(
            dimension_semantics=("parallel",)),
    )(u_rows, m_rows, w1u_t, w1m_t, b1, w2, b2)


def kernel(x, user_emb, movie_emb, W1, b1, W2, b2):
    users = x[0].astype(jnp.int32).reshape(_NW, _NCH, _CHUNK)
    movies = x[1].astype(jnp.int32).reshape(_NW, _NCH, _CHUNK)
    u_rows, m_rows = _sc_gather(users, movies, user_emb, movie_emb)
    w1u_t = W1[:, :_D].T
    w1m_t = W1[:, _D:].T
    return _tc_mlp(u_rows, m_rows, w1u_t, w1m_t,
                   b1.reshape(1, _H1), W2, b2.reshape(1, 1))


# speedup vs baseline: 2.3107x; 2.1072x over previous
import functools

import jax
import jax.numpy as jnp
from jax import lax
from jax.experimental import pallas as pl
from jax.experimental.pallas import tpu as pltpu
from jax.experimental.pallas import tpu_sc as plsc

_B = 16384
_D = 64
_V = 100000
_H1 = 256
_COLS_PER_TILE = _D // 16
_HALF = _B // 2

_CBLK = 2048


def _sc_gather_t(uidx, midx, uemb_t, memb_t):
    mesh = plsc.VectorSubcoreMesh(core_axis_name="c", subcore_axis_name="s")

    @functools.partial(
        pl.kernel,
        mesh=mesh,
        out_type=[
            jax.ShapeDtypeStruct((_D, _B), jnp.float32),
            jax.ShapeDtypeStruct((_D, _B), jnp.float32),
        ],
        scratch_types=[
            pltpu.VMEM((_V,), jnp.float32),
            pltpu.VMEM((_B,), jnp.int32),
            pltpu.VMEM((_HALF,), jnp.float32),
        ],
        compiler_params=pltpu.CompilerParams(
            use_tc_tiling_on_sc=True, needs_layout_passes=False),
    )
    def k(uidx_hbm, midx_hbm, uemb_hbm, memb_hbm, out_u, out_m,
          col_v, idx_v, row_v):
        wid = lax.axis_index("s") * 2 + lax.axis_index("c")
        slot = wid // 2
        is_user = (wid % 2) == 0

        def work(idx_hbm, tab_hbm, out_hbm):
            pltpu.sync_copy(idx_hbm, idx_v)
            for j in range(_COLS_PER_TILE):
                c = slot * _COLS_PER_TILE + j
                pltpu.sync_copy(tab_hbm.at[c], col_v)
                for h in range(2):
                    def body(i, _):
                        iv = idx_v[pl.ds(h * _HALF + i * 16, 16)]
                        row_v[pl.ds(i * 16, 16)] = plsc.load_gather(
                            col_v, [iv])
                        return 0
                    lax.fori_loop(0, _HALF // 16, body, 0)
                    pltpu.sync_copy(
                        row_v, out_hbm.at[c, pl.ds(h * _HALF, _HALF)])

        @pl.when(is_user)
        def _():
            work(uidx_hbm, uemb_hbm, out_u)

        @pl.when(jnp.logical_not(is_user))
        def _():
            work(midx_hbm, memb_hbm, out_m)

    return k(uidx, midx, uemb_t, memb_t)


def _mlp_body(u_ref, m_ref, w1_ref, b1_ref, w2_ref, b2_ref, out_ref):
    w1u = w1_ref[:, :_D]
    w1m = w1_ref[:, _D:]
    h = (jnp.dot(w1u, u_ref[...], preferred_element_type=jnp.float32)
         + jnp.dot(w1m, m_ref[...], preferred_element_type=jnp.float32)
         + b1_ref[...])
    h = jnp.maximum(h, 0.0)
    res = jnp.dot(w2_ref[...], h, preferred_element_type=jnp.float32)
    out_ref[...] = res[0, :] + b2_ref[0, 0]


def _tc_mlp(u_t, m_t, W1, b1_2d, W2, b2_2d):
    grid = (_B // _CBLK,)
    return pl.pallas_call(
        _mlp_body,
        grid=grid,
        in_specs=[
            pl.BlockSpec((_D, _CBLK), lambda i: (0, i)),
            pl.BlockSpec((_D, _CBLK), lambda i: (0, i)),
            pl.BlockSpec((_H1, 2 * _D), lambda i: (0, 0)),
            pl.BlockSpec((_H1, 1), lambda i: (0, 0)),
            pl.BlockSpec((1, _H1), lambda i: (0, 0)),
            pl.BlockSpec((1, 1), lambda i: (0, 0)),
        ],
        out_specs=pl.BlockSpec((_CBLK,), lambda i: (i,)),
        out_shape=jax.ShapeDtypeStruct((_B,), jnp.float32),
        compiler_params=pltpu.CompilerParams(
            dimension_semantics=("parallel",)),
    )(u_t, m_t, W1, b1_2d, W2, b2_2d)


def kernel(x, user_emb, movie_emb, W1, b1, W2, b2):
    uidx = x[0].astype(jnp.int32)
    midx = x[1].astype(jnp.int32)
    u_t, m_t = _sc_gather_t(uidx, midx, user_emb.T, movie_emb.T)
    return _tc_mlp(u_t, m_t, W1, b1.reshape(_H1, 1), W2, b2.reshape(1, 1))


# speedup vs baseline: 3.2311x; 1.3984x over previous
import functools

import jax
import jax.numpy as jnp
from jax import lax
from jax.experimental import pallas as pl
from jax.experimental.pallas import tpu as pltpu
from jax.experimental.pallas import tpu_sc as plsc

_B = 16384
_D = 64
_V = 100000
_H1 = 256
_COLS_PER_TILE = _D // 16
_HALF = _B // 2

_CBLK = 2048


def _sc_gather_t(uidx, midx, uemb_t, memb_t):
    mesh = plsc.VectorSubcoreMesh(core_axis_name="c", subcore_axis_name="s")

    @functools.partial(
        pl.kernel,
        mesh=mesh,
        out_type=[
            jax.ShapeDtypeStruct((_D, _B), jnp.float32),
            jax.ShapeDtypeStruct((_D, _B), jnp.float32),
        ],
        scratch_types=[
            pltpu.VMEM((_V,), jnp.float32),
            pltpu.VMEM((_B,), jnp.int32),
            pltpu.VMEM((_HALF,), jnp.float32),
        ],
        compiler_params=pltpu.CompilerParams(
            use_tc_tiling_on_sc=True, needs_layout_passes=False),
    )
    def k(uidx_hbm, midx_hbm, uemb_hbm, memb_hbm, out_u, out_m,
          col_v, idx_v, row_v):
        wid = lax.axis_index("s") * 2 + lax.axis_index("c")
        slot = wid // 2
        is_user = (wid % 2) == 0

        def work(idx_hbm, tab_hbm, out_hbm):
            pltpu.sync_copy(idx_hbm, idx_v)
            for j in range(_COLS_PER_TILE):
                c = slot * _COLS_PER_TILE + j
                pltpu.sync_copy(tab_hbm.at[c], col_v)
                for h in range(2):
                    @plsc.parallel_loop(0, _HALF, 16, unroll=8)
                    def _(i):
                        iv = idx_v[pl.ds(h * _HALF + i, 16)]
                        row_v[pl.ds(i, 16)] = plsc.load_gather(
                            col_v, [iv])
                    pltpu.sync_copy(
                        row_v, out_hbm.at[c, pl.ds(h * _HALF, _HALF)])

        @pl.when(is_user)
        def _():
            work(uidx_hbm, uemb_hbm, out_u)

        @pl.when(jnp.logical_not(is_user))
        def _():
            work(midx_hbm, memb_hbm, out_m)

    return k(uidx, midx, uemb_t, memb_t)


def _mlp_body(u_ref, m_ref, w1_ref, b1_ref, w2_ref, b2_ref, out_ref):
    w1u = w1_ref[:, :_D]
    w1m = w1_ref[:, _D:]
    h = (jnp.dot(w1u, u_ref[...], preferred_element_type=jnp.float32)
         + jnp.dot(w1m, m_ref[...], preferred_element_type=jnp.float32)
         + b1_ref[...])
    h = jnp.maximum(h, 0.0)
    res = jnp.dot(w2_ref[...], h, preferred_element_type=jnp.float32)
    out_ref[...] = res[0, :] + b2_ref[0, 0]


def _tc_mlp(u_t, m_t, W1, b1_2d, W2, b2_2d):
    grid = (_B // _CBLK,)
    return pl.pallas_call(
        _mlp_body,
        grid=grid,
        in_specs=[
            pl.BlockSpec((_D, _CBLK), lambda i: (0, i)),
            pl.BlockSpec((_D, _CBLK), lambda i: (0, i)),
            pl.BlockSpec((_H1, 2 * _D), lambda i: (0, 0)),
            pl.BlockSpec((_H1, 1), lambda i: (0, 0)),
            pl.BlockSpec((1, _H1), lambda i: (0, 0)),
            pl.BlockSpec((1, 1), lambda i: (0, 0)),
        ],
        out_specs=pl.BlockSpec((_CBLK,), lambda i: (i,)),
        out_shape=jax.ShapeDtypeStruct((_B,), jnp.float32),
        compiler_params=pltpu.CompilerParams(
            dimension_semantics=("parallel",)),
    )(u_t, m_t, W1, b1_2d, W2, b2_2d)


def kernel(x, user_emb, movie_emb, W1, b1, W2, b2):
    uidx = x[0].astype(jnp.int32)
    midx = x[1].astype(jnp.int32)
    u_t, m_t = _sc_gather_t(uidx, midx, user_emb.T, movie_emb.T)
    return _tc_mlp(u_t, m_t, W1, b1.reshape(_H1, 1), W2, b2.reshape(1, 1))
